# baseline (speedup 1.0000x reference)
import functools

import jax
import jax.numpy as jnp
from jax import lax
from jax.experimental import pallas as pl
from jax.experimental.pallas import tpu as pltpu
from jax.experimental.pallas import tpu_sc as plsc

_EP_PAD = 17


def _sc_loss(B, L, E):
    NG = L // 16
    EP = E + _EP_PAD
    E16 = E * 16
    n_pairs = float(B * (L * (L - 1)) // 2)
    mesh = plsc.VectorSubcoreMesh(
        core_axis_name="c", subcore_axis_name="s", num_cores=1)

    @functools.partial(
        pl.kernel,
        mesh=mesh,
        out_type=jax.ShapeDtypeStruct((16,), jnp.float32),
        compiler_params=pltpu.CompilerParams(needs_layout_passes=False),
        scratch_types=[
            pltpu.VMEM((L * E,), jnp.float32),
            pltpu.VMEM((4 * E + L + 16,), jnp.float32),
            pltpu.VMEM((2, L), jnp.float32),
            pltpu.VMEM((16,), jnp.float32),
            pltpu.SMEM((1,), jnp.int32),
            pltpu.SemaphoreType.DMA,
        ],
    )
    def k(enc_hbm, small_hbm, out_hbm,
          enc_v, small_v, a_v, out_v, acc_smem, sem):
        s = lax.axis_index("s")
        cp1 = pltpu.async_copy(enc_hbm.at[s], enc_v, sem)
        cp2 = pltpu.async_copy(small_hbm.at[s], small_v, sem)
        cp1.wait()
        cp2.wait()

        lanes = jnp.arange(16, dtype=jnp.int32)
        zi = jnp.zeros((16,), jnp.int32)
        zf = jnp.zeros((16,), jnp.float32)
        NC = E // 16

        grp_vecs = []
        for g_i in range(NG):
            def blk_body(i, carry, _g=g_i):
                a0v, a1v, c0v, c1v = carry
                for r in range(4):
                    jloc = 4 * i + r
                    jrow = jnp.full((16,), (16 * _g) * E, jnp.int32) \
                        + jloc * E + lanes
                    acc0 = acc1 = acc2 = acc3 = zf
                    for c in range(NC):
                        gvec = plsc.load_gather(enc_v, [jrow + 16 * c])
                        w0l = small_v[pl.ds(16 * c, 16)]
                        w1l = small_v[pl.ds(E + 16 * c, 16)]
                        w0r = small_v[pl.ds(2 * E + 16 * c, 16)]
                        w1r = small_v[pl.ds(3 * E + 16 * c, 16)]
                        acc0 = acc0 + gvec * w0l
                        acc1 = acc1 + gvec * w1l
                        acc2 = acc2 + gvec * w0r
                        acc3 = acc3 + gvec * w1r
                    sel = lanes == jloc
                    a0v = jnp.where(sel, jnp.sum(acc0), a0v)
                    a1v = jnp.where(sel, jnp.sum(acc1), a1v)
                    c0v = jnp.where(sel, jnp.sum(acc2), c0v)
                    c1v = jnp.where(sel, jnp.sum(acc3), c1v)
                return a0v, a1v, c0v, c1v

            grp_vecs.append(
                lax.fori_loop(0, 4, blk_body, (zf, zf, zf, zf)))

        msum = zf
        for g_i in range(NG):
            msum = msum + small_v[pl.ds(4 * E + 16 * g_i, 16)]
        tl = jnp.sum(msum)
        bv = small_v[pl.ds(4 * E + L, 16)]
        b0 = bv[0]
        b1 = bv[1]
        c_vecs = []
        for g_i in range(NG):
            jj = (lanes + 16 * g_i).astype(jnp.float32)
            valid = jnp.where(jj < tl, 1.0, 0.0)
            a_v[0, pl.ds(16 * g_i, 16)] = grp_vecs[g_i][0] * valid + b0
            a_v[1, pl.ds(16 * g_i, 16)] = grp_vecs[g_i][1] * valid + b1
            c_vecs.append((grp_vecs[g_i][2] * valid,
                           grp_vecs[g_i][3] * valid))

        def make_pair_body(n_chunks):
            def pair_body(j, acc):
                jcol = jnp.full((16,), j, dtype=jnp.int32)
                a0bc = plsc.load_gather(a_v, [zi, jcol])
                a1bc = plsc.load_gather(a_v, [zi + 1, jcol])
                total = acc
                for g_i in range(n_chunks):
                    c0, c1 = c_vecs[g_i]
                    kk = lanes + 16 * g_i
                    x0 = a0bc + c0
                    x1 = a1bc + c1
                    m = jnp.maximum(x0, x1)
                    d = jnp.abs(x0 - x1)
                    z = jnp.exp(-d)
                    w = z / (2.0 + z)
                    w2 = w * w
                    t = w2 * (1.0 / 3.0
                              + w2 * (1.0 / 5.0 + w2 * (1.0 / 7.0)))
                    lse = m + 2.0 * (w + w * t)
                    pick = jnp.where(kk == j - 1, x1, x0)
                    total = total + jnp.where(kk < j, pick - lse, 0.0)
                return total
            return pair_body

        part = zf
        for seg in range(NG):
            part = lax.fori_loop(16 * seg, 16 * (seg + 1),
                                 make_pair_body(seg + 1), part)

        acc_smem[0] = 0
        plsc.subcore_barrier()
        q = (jnp.sum(part) * 1024.0).astype(jnp.int32)
        plsc.fetch_and_add(acc_smem.at[0], q, subcore_id=jnp.int32(0))
        plsc.subcore_barrier()

        @pl.when(s == 0)
        def _():
            tot = acc_smem[0].astype(jnp.float32)
            loss = tot * (-1.0 / (1024.0 * n_pairs))
            out_v[...] = jnp.full((16,), loss)
            pltpu.sync_copy(out_v, out_hbm)

    return k


def kernel(encoder_output, mask, W, b):
    B, L, E = encoder_output.shape
    wt = jnp.concatenate([W[:, :E], W[:, E:]], axis=0)
    enc_flat = encoder_output.reshape(B, L * E)
    bpad = jnp.zeros((16,), W.dtype).at[0:2].set(b)
    small = jnp.concatenate(
        [jnp.broadcast_to(wt.reshape(4 * E), (B, 4 * E)), mask,
         jnp.broadcast_to(bpad, (B, 16))], axis=1)
    out = _sc_loss(B, L, E)(enc_flat, small)
    return out[0]


# speedup vs baseline: 1.3589x; 1.3589x over previous
import functools

import jax
import jax.numpy as jnp
from jax import lax
from jax.experimental import pallas as pl
from jax.experimental.pallas import tpu as pltpu
from jax.experimental.pallas import tpu_sc as plsc

_EP_PAD = 17


def _sc_loss(B, L, E):
    NG = L // 16
    EP = E + _EP_PAD
    n_pairs = float(B * (L * (L - 1)) // 2)
    mesh = plsc.VectorSubcoreMesh(
        core_axis_name="c", subcore_axis_name="s", num_cores=1)

    @functools.partial(
        pl.kernel,
        mesh=mesh,
        out_type=jax.ShapeDtypeStruct((16,), jnp.float32),
        compiler_params=pltpu.CompilerParams(needs_layout_passes=False),
        scratch_types=[
            pltpu.VMEM((L * EP,), jnp.float32),
            pltpu.VMEM((4 * E + L + 16,), jnp.float32),
            pltpu.VMEM((2, L), jnp.float32),
            pltpu.VMEM((16,), jnp.float32),
            pltpu.SMEM((1,), jnp.int32),
            pltpu.SemaphoreType.DMA,
        ],
    )
    def k(enc_hbm, small_hbm, out_hbm,
          enc_v, small_v, a_v, out_v, acc_smem, sem):
        s = lax.axis_index("s")
        cp1 = pltpu.async_copy(enc_hbm.at[s], enc_v, sem)
        cp2 = pltpu.async_copy(small_hbm.at[s], small_v, sem)
        cp1.wait()
        cp2.wait()

        lanes = jnp.arange(16, dtype=jnp.int32)
        zi = jnp.zeros((16,), jnp.int32)
        zf = jnp.zeros((16,), jnp.float32)
        bases = [(lanes + 16 * g_i) * EP for g_i in range(NG)]
        l4 = lanes & 3

        UNR = 2

        def dot_body(i, accs):
            accs = list(accs)
            e0 = i * UNR
            for u in range(UNR):
                e = e0 + u
                ecol = jnp.full((16,), e, dtype=jnp.int32)
                wvec = plsc.load_gather(small_v, [l4 + e * 4])
                w0l = wvec.at[zi].get(mode="promise_in_bounds")
                w1l = wvec.at[zi + 1].get(mode="promise_in_bounds")
                w0r = wvec.at[zi + 2].get(mode="promise_in_bounds")
                w1r = wvec.at[zi + 3].get(mode="promise_in_bounds")
                for g_i in range(NG):
                    a0, a1, c0, c1 = accs[4 * g_i:4 * g_i + 4]
                    g = plsc.load_gather(enc_v, [bases[g_i] + ecol])
                    accs[4 * g_i:4 * g_i + 4] = (
                        a0 + g * w0l, a1 + g * w1l,
                        c0 + g * w0r, c1 + g * w1r)
            return tuple(accs)

        accs = lax.fori_loop(0, E // UNR, dot_body, (zf,) * (4 * NG))

        msum = zf
        for g_i in range(NG):
            msum = msum + small_v[pl.ds(4 * E + 16 * g_i, 16)]
        tl = jnp.sum(msum)
        bv = small_v[pl.ds(4 * E + L, 16)]
        b0 = bv[0]
        b1 = bv[1]
        c_vecs = []
        for g_i in range(NG):
            jj = (lanes + 16 * g_i).astype(jnp.float32)
            valid = jnp.where(jj < tl, 1.0, 0.0)
            a_v[0, pl.ds(16 * g_i, 16)] = accs[4 * g_i + 0] * valid + b0
            a_v[1, pl.ds(16 * g_i, 16)] = accs[4 * g_i + 1] * valid + b1
            c_vecs.append((accs[4 * g_i + 2] * valid,
                           accs[4 * g_i + 3] * valid))

        def make_pair_body(n_chunks):
            def pair_body(j, acc):
                jcol = jnp.full((16,), j, dtype=jnp.int32)
                a0bc = plsc.load_gather(a_v, [zi, jcol])
                a1bc = plsc.load_gather(a_v, [zi + 1, jcol])
                total = acc
                for g_i in range(n_chunks):
                    c0, c1 = c_vecs[g_i]
                    kk = lanes + 16 * g_i
                    x0 = a0bc + c0
                    x1 = a1bc + c1
                    m = jnp.maximum(x0, x1)
                    d = jnp.abs(x0 - x1)
                    z = jnp.exp(-d)
                    w = z / (2.0 + z)
                    w2 = w * w
                    t = w2 * (1.0 / 3.0
                              + w2 * (1.0 / 5.0 + w2 * (1.0 / 7.0)))
                    lse = m + 2.0 * (w + w * t)
                    pick = jnp.where(kk == j - 1, x1, x0)
                    total = total + jnp.where(kk < j, pick - lse, 0.0)
                return total
            return pair_body

        part = zf
        for seg in range(NG):
            part = lax.fori_loop(16 * seg, 16 * (seg + 1),
                                 make_pair_body(seg + 1), part)

        acc_smem[0] = 0
        plsc.subcore_barrier()
        q = (jnp.sum(part) * 256.0).astype(jnp.int32)
        plsc.fetch_and_add(acc_smem.at[0], q, subcore_id=jnp.int32(0))
        plsc.subcore_barrier()

        @pl.when(s == 0)
        def _():
            tot = acc_smem[0].astype(jnp.float32)
            loss = tot * (-1.0 / (256.0 * n_pairs))
            out_v[...] = jnp.full((16,), loss)
            pltpu.sync_copy(out_v, out_hbm)

    return k


def kernel(encoder_output, mask, W, b):
    B, L, E = encoder_output.shape
    wt = jnp.concatenate([W[:, :E], W[:, E:]], axis=0)
    wq = wt.T.reshape(4 * E)
    EP = E + _EP_PAD
    enc_pad = jnp.pad(encoder_output, ((0, 0), (0, 0), (0, EP - E)))
    enc_flat = enc_pad.reshape(B, L * EP)
    bpad = jnp.zeros((16,), W.dtype).at[0:2].set(b)
    small = jnp.concatenate(
        [jnp.broadcast_to(wq, (B, 4 * E)), mask,
         jnp.broadcast_to(bpad, (B, 16))], axis=1)
    out = _sc_loss(B, L, E)(enc_flat, small)
    return out[0]
